# baseline (speedup 1.0000x reference)
import functools

import jax
import jax.numpy as jnp
from jax import lax
from jax.experimental import pallas as pl
from jax.experimental.pallas import tpu as pltpu
from jax.experimental.pallas import tpu_sc as plsc

BLK = 128
TB = 256
E = 8
K = 2
NW = 32


def _router_shared_body(x16_ref, rw_ref, sg_ref, su_ref, sd_ref, seg_ref,
                        egw_ref, euw_ref, edw_ref, sgx_ref, sux_ref, sdx_ref,
                        sh_ref, m_ref, gq_ref, uq_ref, dq_ref):
    t = pl.program_id(0)
    D = x16_ref.shape[1]
    DFF = egw_ref.shape[1]
    for i in range(DFF // BLK):
        ri = slice(i * BLK, (i + 1) * BLK)
        gq_ref[0, ri, :] = (egw_ref[0, ri, :]
                            * sgx_ref[t, i, :]).astype(jnp.bfloat16)
        uq_ref[0, ri, :] = (euw_ref[0, ri, :]
                            * sux_ref[t, i, :]).astype(jnp.bfloat16)
    for i in range(D // BLK):
        ri = slice(i * BLK, (i + 1) * BLK)
        dq_ref[0, ri, :] = (edw_ref[0, ri, :]
                            * sdx_ref[t, i, :]).astype(jnp.bfloat16)

    xb = x16_ref[...]
    logits = jax.lax.dot_general(
        xb, rw_ref[...], (((1,), (1,)), ((), ())),
        preferred_element_type=jnp.float32)
    mx = jnp.max(logits, axis=1, keepdims=True)
    ex = jnp.exp(logits - mx)
    p = ex / jnp.sum(ex, axis=1, keepdims=True)
    lane = jax.lax.broadcasted_iota(jnp.int32, p.shape, 1).astype(jnp.float32)
    m1 = jnp.max(p, axis=1, keepdims=True)
    i1 = jnp.min(jnp.where(p == m1, lane, jnp.float32(1e9)), axis=1,
                 keepdims=True)
    p2 = jnp.where(lane == i1, jnp.float32(-1e30), p)
    m2 = jnp.max(p2, axis=1, keepdims=True)
    i2 = jnp.min(jnp.where(p2 == m2, lane, jnp.float32(1e9)), axis=1,
                 keepdims=True)
    den = m1 + m2
    m_ref[:, 0:1] = i1
    m_ref[:, 1:2] = i2
    m_ref[:, 2:3] = m1 / den
    m_ref[:, 3:4] = m2 / den

    sgm = jax.lax.dot_general(xb, sg_ref[...], (((1,), (1,)), ((), ())),
                              preferred_element_type=jnp.float32)
    sup = jax.lax.dot_general(xb, su_ref[...], (((1,), (1,)), ((), ())),
                              preferred_element_type=jnp.float32)
    sh = (sgm * jax.nn.sigmoid(sgm) * sup).astype(jnp.bfloat16)
    shared = jax.lax.dot_general(sh, sd_ref[...], (((1,), (1,)), ((), ())),
                                 preferred_element_type=jnp.float32)
    glog = jnp.sum(xb.astype(jnp.float32) * seg_ref[...], axis=1,
                   keepdims=True)
    sh_ref[...] = jax.nn.sigmoid(glog) * shared


def _ffn_body(beid_ref, bact_ref, xs_ref, ws_ref, gq_ref, uq_ref, dq_ref,
              ys_ref):
    b = pl.program_id(0)
    act = bact_ref[b] != 0

    @pl.when(act)
    def _compute():
        xb = xs_ref[...].astype(jnp.bfloat16)
        gate = jax.lax.dot_general(xb, gq_ref[0], (((1,), (1,)), ((), ())),
                                   preferred_element_type=jnp.float32)
        up = jax.lax.dot_general(xb, uq_ref[0], (((1,), (1,)), ((), ())),
                                 preferred_element_type=jnp.float32)
        h = (gate * jax.nn.sigmoid(gate) * up).astype(jnp.bfloat16)
        y = jax.lax.dot_general(h, dq_ref[0], (((1,), (1,)), ((), ())),
                                preferred_element_type=jnp.float32)
        ys_ref[...] = y * ws_ref[:, 0:1]


def _make_dispatch(T, D, A_MAX):
    tpw = T // NW

    @functools.partial(
        pl.kernel,
        mesh=plsc.VectorSubcoreMesh(core_axis_name="c", subcore_axis_name="s"),
        out_type=[
            jax.ShapeDtypeStruct((A_MAX, D), jnp.float32),
            jax.ShapeDtypeStruct((A_MAX, 128), jnp.float32),
        ],
        scratch_types=[
            pltpu.VMEM((tpw, D), jnp.float32),
            pltpu.VMEM((tpw, 128), jnp.float32),
            pltpu.VMEM((tpw, 128), jnp.float32),
            pltpu.VMEM((tpw,), jnp.int32),
            pltpu.VMEM((tpw,), jnp.int32),
            pltpu.SemaphoreType.DMA,
        ],
    )
    def dispatch(x_hbm, s0_hbm, s1_hbm, w0_hbm, w1_hbm, xs_hbm, ws_hbm,
                 rows_v, w0_v, w1_v, i0_v, i1_v, sem):
        wid = lax.axis_index("s") * 2 + lax.axis_index("c")
        base = wid * tpw
        pltpu.sync_copy(x_hbm.at[pl.ds(base, tpw)], rows_v)
        pltpu.sync_copy(s0_hbm.at[pl.ds(base, tpw)], i0_v)
        pltpu.sync_copy(s1_hbm.at[pl.ds(base, tpw)], i1_v)
        pltpu.sync_copy(w0_hbm.at[pl.ds(base, tpw)], w0_v)
        pltpu.sync_copy(w1_hbm.at[pl.ds(base, tpw)], w1_v)
        cps = [
            pltpu.async_copy(rows_v, xs_hbm.at[i0_v], sem),
            pltpu.async_copy(rows_v, xs_hbm.at[i1_v], sem),
            pltpu.async_copy(w0_v, ws_hbm.at[i0_v], sem),
            pltpu.async_copy(w1_v, ws_hbm.at[i1_v], sem),
        ]
        for cp in cps:
            cp.wait()

    return dispatch


def _make_combine(T, D, A_MAX):
    tpw = T // NW
    CH = 16
    NCH = tpw // CH

    @functools.partial(
        pl.kernel,
        mesh=plsc.VectorSubcoreMesh(core_axis_name="c", subcore_axis_name="s"),
        out_type=jax.ShapeDtypeStruct((T, D), jnp.float32),
        scratch_types=[
            pltpu.VMEM((CH, D), jnp.float32),
            pltpu.VMEM((CH, D), jnp.float32),
            pltpu.VMEM((CH, D), jnp.float32),
            pltpu.VMEM((CH,), jnp.int32),
            pltpu.VMEM((CH,), jnp.int32),
            pltpu.SemaphoreType.DMA,
        ],
    )
    def combine(ys_hbm, sh_hbm, s0_hbm, s1_hbm, out_hbm,
                g0_v, g1_v, acc_v, i0_v, i1_v, sem):
        wid = lax.axis_index("s") * 2 + lax.axis_index("c")
        base = wid * tpw

        def chunk(ci, carry):
            cbase = base + ci * CH
            pltpu.sync_copy(s0_hbm.at[pl.ds(cbase, CH)], i0_v)
            pltpu.sync_copy(s1_hbm.at[pl.ds(cbase, CH)], i1_v)
            pltpu.sync_copy(sh_hbm.at[pl.ds(cbase, CH)], acc_v)
            cp0 = pltpu.async_copy(ys_hbm.at[i0_v], g0_v, sem)
            cp1 = pltpu.async_copy(ys_hbm.at[i1_v], g1_v, sem)
            cp0.wait()
            cp1.wait()

            def tstep(t, carry2):
                for d in range(D // 16):
                    sl = pl.ds(d * 16, 16)
                    acc_v[t, sl] = acc_v[t, sl] + g0_v[t, sl] + g1_v[t, sl]
                return carry2

            lax.fori_loop(0, CH, tstep, 0)
            pltpu.sync_copy(acc_v, out_hbm.at[pl.ds(cbase, CH)])
            return carry

        lax.fori_loop(0, NCH, chunk, 0)

    return combine


def kernel(hidden_states, router_w, shared_gate_w, shared_up_w, shared_down_w,
           shared_expert_gate_w, eg_w, eg_s, eu_w, eu_s, ed_w, ed_s):
    bsz, seq, D = hidden_states.shape
    T = bsz * seq
    _, DFF, _ = eg_w.shape
    DSH = shared_gate_w.shape[0]
    NT = T // min(256, T)
    A_MAX = K * T + E * TB
    NB = A_MAX // TB

    x = hidden_states.reshape(T, D)
    x16 = x.astype(jnp.bfloat16)
    rw16 = router_w.astype(jnp.bfloat16)
    sg16 = shared_gate_w.astype(jnp.bfloat16)
    su16 = shared_up_w.astype(jnp.bfloat16)
    sd16 = shared_down_w.astype(jnp.bfloat16)
    seg16 = shared_expert_gate_w.astype(jnp.bfloat16)
    sgx = jnp.repeat(eg_s, BLK, axis=2)
    sux = jnp.repeat(eu_s, BLK, axis=2)
    sdx = jnp.repeat(ed_s, BLK, axis=2)

    TBA = T // NT
    sh_out, m, gq16, uq16, dq16 = pl.pallas_call(
        _router_shared_body,
        grid=(NT,),
        in_specs=[
            pl.BlockSpec((TBA, D), lambda t: (t, 0)),
            pl.BlockSpec((E, D), lambda t: (0, 0)),
            pl.BlockSpec((DSH, D), lambda t: (0, 0)),
            pl.BlockSpec((DSH, D), lambda t: (0, 0)),
            pl.BlockSpec((D, DSH), lambda t: (0, 0)),
            pl.BlockSpec((1, D), lambda t: (0, 0)),
            pl.BlockSpec((1, DFF, D), lambda t: (t, 0, 0)),
            pl.BlockSpec((1, DFF, D), lambda t: (t, 0, 0)),
            pl.BlockSpec((1, D, DFF), lambda t: (t, 0, 0)),
            pl.BlockSpec((E, DFF // BLK, D), lambda t: (0, 0, 0)),
            pl.BlockSpec((E, DFF // BLK, D), lambda t: (0, 0, 0)),
            pl.BlockSpec((E, D // BLK, DFF), lambda t: (0, 0, 0)),
        ],
        out_specs=[
            pl.BlockSpec((TBA, D), lambda t: (t, 0)),
            pl.BlockSpec((TBA, 8), lambda t: (t, 0)),
            pl.BlockSpec((1, DFF, D), lambda t: (t, 0, 0)),
            pl.BlockSpec((1, DFF, D), lambda t: (t, 0, 0)),
            pl.BlockSpec((1, D, DFF), lambda t: (t, 0, 0)),
        ],
        out_shape=[
            jax.ShapeDtypeStruct((T, D), jnp.float32),
            jax.ShapeDtypeStruct((T, 8), jnp.float32),
            jax.ShapeDtypeStruct((E, DFF, D), jnp.bfloat16),
            jax.ShapeDtypeStruct((E, DFF, D), jnp.bfloat16),
            jax.ShapeDtypeStruct((E, D, DFF), jnp.bfloat16),
        ],
    )(x16, rw16, sg16, su16, sd16, seg16, eg_w, eu_w, ed_w, sgx, sux, sdx)

    i1 = m[:, 0].astype(jnp.int32)
    i2 = m[:, 1].astype(jnp.int32)
    eid = jnp.stack([i1, i2], axis=1).reshape(-1)
    oh = (eid[:, None] == jnp.arange(E)[None, :]).astype(jnp.int32)
    pos_incl = jnp.cumsum(oh, axis=0)
    pos = jnp.sum(oh * pos_incl, axis=1) - 1
    counts = pos_incl[-1]
    nblk = (counts + TB - 1) // TB
    cumblk = jnp.cumsum(nblk)
    blk_start = cumblk - nblk
    slot = jnp.sum(oh * blk_start[None, :], axis=1) * TB + pos
    bidx = jnp.arange(NB, dtype=jnp.int32)
    beid_raw = (bidx[:, None] >= cumblk[None, :]).sum(axis=1).astype(jnp.int32)
    beid = jnp.minimum(beid_raw, E - 1)
    bact = (beid_raw < E).astype(jnp.int32)

    s_tk = slot.reshape(T, K)
    s0 = s_tk[:, 0]
    s1 = s_tk[:, 1]
    w0r = jnp.broadcast_to(m[:, 2:3], (T, 128))
    w1r = jnp.broadcast_to(m[:, 3:4], (T, 128))

    xs, ws = _make_dispatch(T, D, A_MAX)(x, s0, s1, w0r, w1r)

    ys = pl.pallas_call(
        _ffn_body,
        grid_spec=pltpu.PrefetchScalarGridSpec(
            num_scalar_prefetch=2,
            grid=(NB,),
            in_specs=[
                pl.BlockSpec((TB, D), lambda b, beid, bact: (bact[b] * b, 0)),
                pl.BlockSpec((TB, 128),
                             lambda b, beid, bact: (bact[b] * b, 0)),
                pl.BlockSpec((1, DFF, D),
                             lambda b, beid, bact: (beid[b], 0, 0)),
                pl.BlockSpec((1, DFF, D),
                             lambda b, beid, bact: (beid[b], 0, 0)),
                pl.BlockSpec((1, D, DFF),
                             lambda b, beid, bact: (beid[b], 0, 0)),
            ],
            out_specs=pl.BlockSpec((TB, D), lambda b, beid, bact: (b, 0)),
        ),
        out_shape=jax.ShapeDtypeStruct((A_MAX, D), jnp.float32),
    )(beid, bact, xs, ws, gq16, uq16, dq16)

    out = _make_combine(T, D, A_MAX)(ys, sh_out, s0, s1)
    return out.reshape(bsz, seq, D)


# speedup vs baseline: 1.0975x; 1.0975x over previous
import functools

import jax
import jax.numpy as jnp
from jax import lax
from jax.experimental import pallas as pl
from jax.experimental.pallas import tpu as pltpu
from jax.experimental.pallas import tpu_sc as plsc

BLK = 128
TB = 256
E = 8
K = 2
NW = 32


def _router_shared_body(x16_ref, rw_ref, sg_ref, su_ref, sd_ref, seg_ref,
                        sh_ref, m_ref):
    xb = x16_ref[...]
    logits = jax.lax.dot_general(
        xb, rw_ref[...], (((1,), (1,)), ((), ())),
        preferred_element_type=jnp.float32)
    mx = jnp.max(logits, axis=1, keepdims=True)
    ex = jnp.exp(logits - mx)
    p = ex / jnp.sum(ex, axis=1, keepdims=True)
    lane = jax.lax.broadcasted_iota(jnp.int32, p.shape, 1).astype(jnp.float32)
    m1 = jnp.max(p, axis=1, keepdims=True)
    i1 = jnp.min(jnp.where(p == m1, lane, jnp.float32(1e9)), axis=1,
                 keepdims=True)
    p2 = jnp.where(lane == i1, jnp.float32(-1e30), p)
    m2 = jnp.max(p2, axis=1, keepdims=True)
    i2 = jnp.min(jnp.where(p2 == m2, lane, jnp.float32(1e9)), axis=1,
                 keepdims=True)
    den = m1 + m2
    m_ref[:, 0:1] = i1
    m_ref[:, 1:2] = i2
    m_ref[:, 2:3] = m1 / den
    m_ref[:, 3:4] = m2 / den

    sgm = jax.lax.dot_general(xb, sg_ref[...], (((1,), (1,)), ((), ())),
                              preferred_element_type=jnp.float32)
    sup = jax.lax.dot_general(xb, su_ref[...], (((1,), (1,)), ((), ())),
                              preferred_element_type=jnp.float32)
    sh = (sgm * jax.nn.sigmoid(sgm) * sup).astype(jnp.bfloat16)
    shared = jax.lax.dot_general(sh, sd_ref[...], (((1,), (1,)), ((), ())),
                                 preferred_element_type=jnp.float32)
    glog = jnp.sum(xb.astype(jnp.float32) * seg_ref[...], axis=1,
                   keepdims=True)
    sh_ref[...] = jax.nn.sigmoid(glog) * shared


def _ffn_body(beid_ref, bact_ref, xs_ref, ws_ref, egw_ref, euw_ref, edw_ref,
              sgx_ref, sux_ref, sdx_ref, ys_ref, gq_ref, uq_ref, dq_ref):
    D = xs_ref.shape[1]
    DFF = egw_ref.shape[1]
    b = pl.program_id(0)
    e = beid_ref[b]
    act = bact_ref[b] != 0

    prev_e = jnp.where(b > 0, beid_ref[jnp.maximum(b - 1, 0)], -1)

    @pl.when((e != prev_e) & act)
    def _dequant():
        for i in range(DFF // BLK):
            ri = slice(i * BLK, (i + 1) * BLK)
            gq_ref[ri, :] = (egw_ref[0, ri, :]
                             * sgx_ref[e, i, :]).astype(jnp.bfloat16)
            uq_ref[ri, :] = (euw_ref[0, ri, :]
                             * sux_ref[e, i, :]).astype(jnp.bfloat16)
        for i in range(D // BLK):
            ri = slice(i * BLK, (i + 1) * BLK)
            dq_ref[ri, :] = (edw_ref[0, ri, :]
                             * sdx_ref[e, i, :]).astype(jnp.bfloat16)

    @pl.when(act)
    def _compute():
        xb = xs_ref[...].astype(jnp.bfloat16)
        gate = jax.lax.dot_general(xb, gq_ref[...], (((1,), (1,)), ((), ())),
                                   preferred_element_type=jnp.float32)
        up = jax.lax.dot_general(xb, uq_ref[...], (((1,), (1,)), ((), ())),
                                 preferred_element_type=jnp.float32)
        h = (gate * jax.nn.sigmoid(gate) * up).astype(jnp.bfloat16)
        y = jax.lax.dot_general(h, dq_ref[...], (((1,), (1,)), ((), ())),
                                preferred_element_type=jnp.float32)
        ys_ref[...] = y * ws_ref[:, 0:1]


def _make_dispatch(T, D, A_MAX):
    tpw = T // NW

    @functools.partial(
        pl.kernel,
        mesh=plsc.VectorSubcoreMesh(core_axis_name="c", subcore_axis_name="s"),
        out_type=[
            jax.ShapeDtypeStruct((A_MAX, D), jnp.float32),
            jax.ShapeDtypeStruct((A_MAX, 128), jnp.float32),
        ],
        scratch_types=[
            pltpu.VMEM((tpw, D), jnp.float32),
            pltpu.VMEM((tpw, 128), jnp.float32),
            pltpu.VMEM((tpw, 128), jnp.float32),
            pltpu.VMEM((tpw,), jnp.int32),
            pltpu.VMEM((tpw,), jnp.int32),
            pltpu.SemaphoreType.DMA,
        ],
    )
    def dispatch(x_hbm, s0_hbm, s1_hbm, w0_hbm, w1_hbm, xs_hbm, ws_hbm,
                 rows_v, w0_v, w1_v, i0_v, i1_v, sem):
        wid = lax.axis_index("s") * 2 + lax.axis_index("c")
        base = wid * tpw
        pltpu.sync_copy(x_hbm.at[pl.ds(base, tpw)], rows_v)
        pltpu.sync_copy(s0_hbm.at[pl.ds(base, tpw)], i0_v)
        pltpu.sync_copy(s1_hbm.at[pl.ds(base, tpw)], i1_v)
        pltpu.sync_copy(w0_hbm.at[pl.ds(base, tpw)], w0_v)
        pltpu.sync_copy(w1_hbm.at[pl.ds(base, tpw)], w1_v)
        cps = [
            pltpu.async_copy(rows_v, xs_hbm.at[i0_v], sem),
            pltpu.async_copy(rows_v, xs_hbm.at[i1_v], sem),
            pltpu.async_copy(w0_v, ws_hbm.at[i0_v], sem),
            pltpu.async_copy(w1_v, ws_hbm.at[i1_v], sem),
        ]
        for cp in cps:
            cp.wait()

    return dispatch


def _make_combine(T, D, A_MAX):
    tpw = T // NW
    CH = 32
    NCH = tpw // CH

    @functools.partial(
        pl.kernel,
        mesh=plsc.VectorSubcoreMesh(core_axis_name="c", subcore_axis_name="s"),
        out_type=jax.ShapeDtypeStruct((T, D), jnp.float32),
        scratch_types=[
            pltpu.VMEM((CH, D), jnp.float32),
            pltpu.VMEM((CH, D), jnp.float32),
            pltpu.VMEM((CH, D), jnp.float32),
            pltpu.VMEM((CH,), jnp.int32),
            pltpu.VMEM((CH,), jnp.int32),
            pltpu.SemaphoreType.DMA,
        ],
    )
    def combine(ys_hbm, sh_hbm, s0_hbm, s1_hbm, out_hbm,
                g0_v, g1_v, acc_v, i0_v, i1_v, sem):
        wid = lax.axis_index("s") * 2 + lax.axis_index("c")
        base = wid * tpw

        def chunk(ci, carry):
            cbase = base + ci * CH
            pltpu.sync_copy(s0_hbm.at[pl.ds(cbase, CH)], i0_v)
            pltpu.sync_copy(s1_hbm.at[pl.ds(cbase, CH)], i1_v)
            pltpu.sync_copy(sh_hbm.at[pl.ds(cbase, CH)], acc_v)
            cp0 = pltpu.async_copy(ys_hbm.at[i0_v], g0_v, sem)
            cp1 = pltpu.async_copy(ys_hbm.at[i1_v], g1_v, sem)
            cp0.wait()
            cp1.wait()

            def tstep(t, carry2):
                for d in range(D // 16):
                    sl = pl.ds(d * 16, 16)
                    acc_v[t, sl] = acc_v[t, sl] + g0_v[t, sl] + g1_v[t, sl]
                return carry2

            lax.fori_loop(0, CH, tstep, 0)
            pltpu.sync_copy(acc_v, out_hbm.at[pl.ds(cbase, CH)])
            return carry

        lax.fori_loop(0, NCH, chunk, 0)

    return combine


def kernel(hidden_states, router_w, shared_gate_w, shared_up_w, shared_down_w,
           shared_expert_gate_w, eg_w, eg_s, eu_w, eu_s, ed_w, ed_s):
    bsz, seq, D = hidden_states.shape
    T = bsz * seq
    _, DFF, _ = eg_w.shape
    DSH = shared_gate_w.shape[0]
    NT = T // min(256, T)
    A_MAX = K * T + E * TB
    NB = A_MAX // TB

    x = hidden_states.reshape(T, D)
    x16 = x.astype(jnp.bfloat16)
    rw16 = router_w.astype(jnp.bfloat16)
    sg16 = shared_gate_w.astype(jnp.bfloat16)
    su16 = shared_up_w.astype(jnp.bfloat16)
    sd16 = shared_down_w.astype(jnp.bfloat16)
    seg16 = shared_expert_gate_w.astype(jnp.bfloat16)
    sgx = jnp.repeat(eg_s, BLK, axis=2)
    sux = jnp.repeat(eu_s, BLK, axis=2)
    sdx = jnp.repeat(ed_s, BLK, axis=2)

    TBA = T // NT
    sh_out, m = pl.pallas_call(
        _router_shared_body,
        grid=(NT,),
        in_specs=[
            pl.BlockSpec((TBA, D), lambda t: (t, 0)),
            pl.BlockSpec((E, D), lambda t: (0, 0)),
            pl.BlockSpec((DSH, D), lambda t: (0, 0)),
            pl.BlockSpec((DSH, D), lambda t: (0, 0)),
            pl.BlockSpec((D, DSH), lambda t: (0, 0)),
            pl.BlockSpec((1, D), lambda t: (0, 0)),
        ],
        out_specs=[
            pl.BlockSpec((TBA, D), lambda t: (t, 0)),
            pl.BlockSpec((TBA, 8), lambda t: (t, 0)),
        ],
        out_shape=[
            jax.ShapeDtypeStruct((T, D), jnp.float32),
            jax.ShapeDtypeStruct((T, 8), jnp.float32),
        ],
    )(x16, rw16, sg16, su16, sd16, seg16)

    i1 = m[:, 0].astype(jnp.int32)
    i2 = m[:, 1].astype(jnp.int32)
    eid = jnp.stack([i1, i2], axis=1).reshape(-1)
    oh = (eid[:, None] == jnp.arange(E)[None, :]).astype(jnp.int32)
    pos_incl = jnp.cumsum(oh, axis=0)
    pos = jnp.sum(oh * pos_incl, axis=1) - 1
    counts = pos_incl[-1]
    nblk = (counts + TB - 1) // TB
    cumblk = jnp.cumsum(nblk)
    blk_start = cumblk - nblk
    slot = jnp.sum(oh * blk_start[None, :], axis=1) * TB + pos
    bidx = jnp.arange(NB, dtype=jnp.int32)
    beid_raw = (bidx[:, None] >= cumblk[None, :]).sum(axis=1).astype(jnp.int32)
    beid = jnp.minimum(beid_raw, E - 1)
    bact = (beid_raw < E).astype(jnp.int32)

    s_tk = slot.reshape(T, K)
    s0 = s_tk[:, 0]
    s1 = s_tk[:, 1]
    w0r = jnp.broadcast_to(m[:, 2:3], (T, 128))
    w1r = jnp.broadcast_to(m[:, 3:4], (T, 128))

    xs, ws = _make_dispatch(T, D, A_MAX)(x, s0, s1, w0r, w1r)

    ys = pl.pallas_call(
        _ffn_body,
        grid_spec=pltpu.PrefetchScalarGridSpec(
            num_scalar_prefetch=2,
            grid=(NB,),
            in_specs=[
                pl.BlockSpec((TB, D), lambda b, beid, bact: (bact[b] * b, 0)),
                pl.BlockSpec((TB, 128),
                             lambda b, beid, bact: (bact[b] * b, 0)),
                pl.BlockSpec((1, DFF, D),
                             lambda b, beid, bact: (beid[b], 0, 0)),
                pl.BlockSpec((1, DFF, D),
                             lambda b, beid, bact: (beid[b], 0, 0)),
                pl.BlockSpec((1, D, DFF),
                             lambda b, beid, bact: (beid[b], 0, 0)),
                pl.BlockSpec((E, DFF // BLK, D),
                             lambda b, beid, bact: (0, 0, 0)),
                pl.BlockSpec((E, DFF // BLK, D),
                             lambda b, beid, bact: (0, 0, 0)),
                pl.BlockSpec((E, D // BLK, DFF),
                             lambda b, beid, bact: (0, 0, 0)),
            ],
            out_specs=pl.BlockSpec((TB, D), lambda b, beid, bact: (b, 0)),
            scratch_shapes=[
                pltpu.VMEM((DFF, D), jnp.bfloat16),
                pltpu.VMEM((DFF, D), jnp.bfloat16),
                pltpu.VMEM((D, DFF), jnp.bfloat16),
            ],
        ),
        out_shape=jax.ShapeDtypeStruct((A_MAX, D), jnp.float32),
    )(beid, bact, xs, ws, eg_w, eu_w, ed_w, sgx, sux, sdx)

    out = _make_combine(T, D, A_MAX)(ys, sh_out, s0, s1)
    return out.reshape(bsz, seq, D)


# speedup vs baseline: 1.1636x; 1.0602x over previous
import functools

import jax
import jax.numpy as jnp
from jax import lax
from jax.experimental import pallas as pl
from jax.experimental.pallas import tpu as pltpu
from jax.experimental.pallas import tpu_sc as plsc

BLK = 128
TB = 512
E = 8
K = 2
NW = 32


def _router_shared_body(x16_ref, rw_ref, sg_ref, su_ref, sd_ref, seg_ref,
                        sh_ref, m_ref):
    xb = x16_ref[...]
    logits = jax.lax.dot_general(
        xb, rw_ref[...], (((1,), (1,)), ((), ())),
        preferred_element_type=jnp.float32)
    mx = jnp.max(logits, axis=1, keepdims=True)
    ex = jnp.exp(logits - mx)
    p = ex / jnp.sum(ex, axis=1, keepdims=True)
    lane = jax.lax.broadcasted_iota(jnp.int32, p.shape, 1).astype(jnp.float32)
    m1 = jnp.max(p, axis=1, keepdims=True)
    i1 = jnp.min(jnp.where(p == m1, lane, jnp.float32(1e9)), axis=1,
                 keepdims=True)
    p2 = jnp.where(lane == i1, jnp.float32(-1e30), p)
    m2 = jnp.max(p2, axis=1, keepdims=True)
    i2 = jnp.min(jnp.where(p2 == m2, lane, jnp.float32(1e9)), axis=1,
                 keepdims=True)
    den = m1 + m2
    m_ref[:, 0:1] = i1
    m_ref[:, 1:2] = i2
    m_ref[:, 2:3] = m1 / den
    m_ref[:, 3:4] = m2 / den

    sgm = jax.lax.dot_general(xb, sg_ref[...], (((1,), (1,)), ((), ())),
                              preferred_element_type=jnp.float32)
    sup = jax.lax.dot_general(xb, su_ref[...], (((1,), (1,)), ((), ())),
                              preferred_element_type=jnp.float32)
    sh = (sgm * jax.nn.sigmoid(sgm) * sup).astype(jnp.bfloat16)
    shared = jax.lax.dot_general(sh, sd_ref[...], (((1,), (1,)), ((), ())),
                                 preferred_element_type=jnp.float32)
    glog = jnp.sum(xb.astype(jnp.float32) * seg_ref[...], axis=1,
                   keepdims=True)
    sh_ref[...] = jax.nn.sigmoid(glog) * shared


def _ffn_body(beid_ref, bact_ref, xs_ref, ws_ref, egw_ref, euw_ref, edw_ref,
              sgx_ref, sux_ref, sdx_ref, ys_ref, gq_ref, uq_ref, dq_ref):
    D = xs_ref.shape[1]
    DFF = egw_ref.shape[1]
    b = pl.program_id(0)
    e = beid_ref[b]
    act = bact_ref[b] != 0

    prev_e = jnp.where(b > 0, beid_ref[jnp.maximum(b - 1, 0)], -1)

    @pl.when((e != prev_e) & act)
    def _dequant():
        for i in range(DFF // BLK):
            ri = slice(i * BLK, (i + 1) * BLK)
            gq_ref[ri, :] = (egw_ref[0, ri, :]
                             * sgx_ref[e, i, :]).astype(jnp.bfloat16)
            uq_ref[ri, :] = (euw_ref[0, ri, :]
                             * sux_ref[e, i, :]).astype(jnp.bfloat16)
        for i in range(D // BLK):
            ri = slice(i * BLK, (i + 1) * BLK)
            dq_ref[ri, :] = (edw_ref[0, ri, :]
                             * sdx_ref[e, i, :]).astype(jnp.bfloat16)

    @pl.when(act)
    def _compute():
        xb = xs_ref[...].astype(jnp.bfloat16)
        gate = jax.lax.dot_general(xb, gq_ref[...], (((1,), (1,)), ((), ())),
                                   preferred_element_type=jnp.float32)
        up = jax.lax.dot_general(xb, uq_ref[...], (((1,), (1,)), ((), ())),
                                 preferred_element_type=jnp.float32)
        h = (gate * jax.nn.sigmoid(gate) * up).astype(jnp.bfloat16)
        y = jax.lax.dot_general(h, dq_ref[...], (((1,), (1,)), ((), ())),
                                preferred_element_type=jnp.float32)
        ys_ref[...] = y * ws_ref[:, 0:1]


def _make_dispatch(T, D, A_MAX):
    tpw = T // NW

    @functools.partial(
        pl.kernel,
        mesh=plsc.VectorSubcoreMesh(core_axis_name="c", subcore_axis_name="s"),
        out_type=[
            jax.ShapeDtypeStruct((A_MAX, D), jnp.float32),
            jax.ShapeDtypeStruct((A_MAX, 128), jnp.float32),
        ],
        scratch_types=[
            pltpu.VMEM((tpw, D), jnp.float32),
            pltpu.VMEM((tpw, 128), jnp.float32),
            pltpu.VMEM((tpw, 128), jnp.float32),
            pltpu.VMEM((tpw,), jnp.int32),
            pltpu.VMEM((tpw,), jnp.int32),
            pltpu.SemaphoreType.DMA,
        ],
    )
    def dispatch(x_hbm, s0_hbm, s1_hbm, w0_hbm, w1_hbm, xs_hbm, ws_hbm,
                 rows_v, w0_v, w1_v, i0_v, i1_v, sem):
        wid = lax.axis_index("s") * 2 + lax.axis_index("c")
        base = wid * tpw
        pltpu.sync_copy(x_hbm.at[pl.ds(base, tpw)], rows_v)
        pltpu.sync_copy(s0_hbm.at[pl.ds(base, tpw)], i0_v)
        pltpu.sync_copy(s1_hbm.at[pl.ds(base, tpw)], i1_v)
        pltpu.sync_copy(w0_hbm.at[pl.ds(base, tpw)], w0_v)
        pltpu.sync_copy(w1_hbm.at[pl.ds(base, tpw)], w1_v)
        cps = [
            pltpu.async_copy(rows_v, xs_hbm.at[i0_v], sem),
            pltpu.async_copy(rows_v, xs_hbm.at[i1_v], sem),
            pltpu.async_copy(w0_v, ws_hbm.at[i0_v], sem),
            pltpu.async_copy(w1_v, ws_hbm.at[i1_v], sem),
        ]
        for cp in cps:
            cp.wait()

    return dispatch


def _make_combine(T, D, A_MAX):
    tpw = T // NW
    CH = 32
    NCH = tpw // CH

    @functools.partial(
        pl.kernel,
        mesh=plsc.VectorSubcoreMesh(core_axis_name="c", subcore_axis_name="s"),
        out_type=jax.ShapeDtypeStruct((T, D), jnp.float32),
        scratch_types=[
            pltpu.VMEM((CH, D), jnp.float32),
            pltpu.VMEM((CH, D), jnp.float32),
            pltpu.VMEM((CH, D), jnp.float32),
            pltpu.VMEM((CH,), jnp.int32),
            pltpu.VMEM((CH,), jnp.int32),
            pltpu.SemaphoreType.DMA,
        ],
    )
    def combine(ys_hbm, sh_hbm, s0_hbm, s1_hbm, out_hbm,
                g0_v, g1_v, acc_v, i0_v, i1_v, sem):
        wid = lax.axis_index("s") * 2 + lax.axis_index("c")
        base = wid * tpw

        def chunk(ci, carry):
            cbase = base + ci * CH
            pltpu.sync_copy(s0_hbm.at[pl.ds(cbase, CH)], i0_v)
            pltpu.sync_copy(s1_hbm.at[pl.ds(cbase, CH)], i1_v)
            pltpu.sync_copy(sh_hbm.at[pl.ds(cbase, CH)], acc_v)
            cp0 = pltpu.async_copy(ys_hbm.at[i0_v], g0_v, sem)
            cp1 = pltpu.async_copy(ys_hbm.at[i1_v], g1_v, sem)
            cp0.wait()
            cp1.wait()

            def tstep(t, carry2):
                for d in range(D // 16):
                    sl = pl.ds(d * 16, 16)
                    acc_v[t, sl] = acc_v[t, sl] + g0_v[t, sl] + g1_v[t, sl]
                return carry2

            lax.fori_loop(0, CH, tstep, 0)
            pltpu.sync_copy(acc_v, out_hbm.at[pl.ds(cbase, CH)])
            return carry

        lax.fori_loop(0, NCH, chunk, 0)

    return combine


def kernel(hidden_states, router_w, shared_gate_w, shared_up_w, shared_down_w,
           shared_expert_gate_w, eg_w, eg_s, eu_w, eu_s, ed_w, ed_s):
    bsz, seq, D = hidden_states.shape
    T = bsz * seq
    _, DFF, _ = eg_w.shape
    DSH = shared_gate_w.shape[0]
    NT = T // min(256, T)
    A_MAX = K * T + E * TB
    NB = A_MAX // TB

    x = hidden_states.reshape(T, D)
    x16 = x.astype(jnp.bfloat16)
    rw16 = router_w.astype(jnp.bfloat16)
    sg16 = shared_gate_w.astype(jnp.bfloat16)
    su16 = shared_up_w.astype(jnp.bfloat16)
    sd16 = shared_down_w.astype(jnp.bfloat16)
    seg16 = shared_expert_gate_w.astype(jnp.bfloat16)
    sgx = jnp.repeat(eg_s, BLK, axis=2)
    sux = jnp.repeat(eu_s, BLK, axis=2)
    sdx = jnp.repeat(ed_s, BLK, axis=2)

    TBA = T // NT
    sh_out, m = pl.pallas_call(
        _router_shared_body,
        grid=(NT,),
        in_specs=[
            pl.BlockSpec((TBA, D), lambda t: (t, 0)),
            pl.BlockSpec((E, D), lambda t: (0, 0)),
            pl.BlockSpec((DSH, D), lambda t: (0, 0)),
            pl.BlockSpec((DSH, D), lambda t: (0, 0)),
            pl.BlockSpec((D, DSH), lambda t: (0, 0)),
            pl.BlockSpec((1, D), lambda t: (0, 0)),
        ],
        out_specs=[
            pl.BlockSpec((TBA, D), lambda t: (t, 0)),
            pl.BlockSpec((TBA, 8), lambda t: (t, 0)),
        ],
        out_shape=[
            jax.ShapeDtypeStruct((T, D), jnp.float32),
            jax.ShapeDtypeStruct((T, 8), jnp.float32),
        ],
    )(x16, rw16, sg16, su16, sd16, seg16)

    i1 = m[:, 0].astype(jnp.int32)
    i2 = m[:, 1].astype(jnp.int32)
    eid = jnp.stack([i1, i2], axis=1).reshape(-1)
    oh = (eid[:, None] == jnp.arange(E)[None, :]).astype(jnp.int32)
    pos_incl = jnp.cumsum(oh, axis=0)
    pos = jnp.sum(oh * pos_incl, axis=1) - 1
    counts = pos_incl[-1]
    nblk = (counts + TB - 1) // TB
    cumblk = jnp.cumsum(nblk)
    blk_start = cumblk - nblk
    slot = jnp.sum(oh * blk_start[None, :], axis=1) * TB + pos
    bidx = jnp.arange(NB, dtype=jnp.int32)
    beid_raw = (bidx[:, None] >= cumblk[None, :]).sum(axis=1).astype(jnp.int32)
    beid = jnp.minimum(beid_raw, E - 1)
    bact = (beid_raw < E).astype(jnp.int32)

    s_tk = slot.reshape(T, K)
    s0 = s_tk[:, 0]
    s1 = s_tk[:, 1]
    w0r = jnp.broadcast_to(m[:, 2:3], (T, 128))
    w1r = jnp.broadcast_to(m[:, 3:4], (T, 128))

    xs, ws = _make_dispatch(T, D, A_MAX)(x, s0, s1, w0r, w1r)

    ys = pl.pallas_call(
        _ffn_body,
        grid_spec=pltpu.PrefetchScalarGridSpec(
            num_scalar_prefetch=2,
            grid=(NB,),
            in_specs=[
                pl.BlockSpec((TB, D), lambda b, beid, bact: (bact[b] * b, 0)),
                pl.BlockSpec((TB, 128),
                             lambda b, beid, bact: (bact[b] * b, 0)),
                pl.BlockSpec((1, DFF, D),
                             lambda b, beid, bact: (beid[b], 0, 0)),
                pl.BlockSpec((1, DFF, D),
                             lambda b, beid, bact: (beid[b], 0, 0)),
                pl.BlockSpec((1, D, DFF),
                             lambda b, beid, bact: (beid[b], 0, 0)),
                pl.BlockSpec((E, DFF // BLK, D),
                             lambda b, beid, bact: (0, 0, 0)),
                pl.BlockSpec((E, DFF // BLK, D),
                             lambda b, beid, bact: (0, 0, 0)),
                pl.BlockSpec((E, D // BLK, DFF),
                             lambda b, beid, bact: (0, 0, 0)),
            ],
            out_specs=pl.BlockSpec((TB, D), lambda b, beid, bact: (b, 0)),
            scratch_shapes=[
                pltpu.VMEM((DFF, D), jnp.bfloat16),
                pltpu.VMEM((DFF, D), jnp.bfloat16),
                pltpu.VMEM((D, DFF), jnp.bfloat16),
            ],
        ),
        out_shape=jax.ShapeDtypeStruct((A_MAX, D), jnp.float32),
    )(beid, bact, xs, ws, eg_w, eu_w, ed_w, sgx, sux, sdx)

    out = _make_combine(T, D, A_MAX)(ys, sh_out, s0, s1)
    return out.reshape(bsz, seq, D)
